# Initial kernel scaffold; baseline (speedup 1.0000x reference)
import functools

import jax
import jax.numpy as jnp
from jax import lax
from jax.experimental import pallas as pl
from jax.experimental.pallas import tpu as pltpu
from jax.experimental.pallas import tpu_sc as plsc

N = 10000
E = 320000
F_IN = 128
H = 128
C = 40
CP = 48

NC = 2
NS = 16
NW = NC * NS
L = 16

NPAD = 10240
K = 128
E2 = E + N
CHUNKS = -(-E2 // (NW * K))
E2P = NW * K * CHUNKS
EPAD = E2P - E2

DEG_K = 80
DEG_CHUNKS = E // (NW * DEG_K)
EPT = E // NW

_mesh = lambda: plsc.VectorSubcoreMesh(core_axis_name="c", subcore_axis_name="s")


def _wid():
    return lax.axis_index("c") * NS + lax.axis_index("s")


def _zero16():
    return jnp.zeros((L,), jnp.float32)


def _deg_body(dst_hbm, w_hbm, out_hbm, deg_s, dstb, wb, zb, sem):
    cid = lax.axis_index("c")
    sid = lax.axis_index("s")
    wid = _wid()

    def _z(i, _):
        zb[pl.ds(i * L, L)] = _zero16()
        return 0
    lax.fori_loop(0, 40, _z, 0)
    pltpu.sync_copy(zb, deg_s.at[pl.ds(sid * 640, 640)])
    plsc.subcore_barrier()

    pltpu.sync_copy(dst_hbm.at[pl.ds(wid * DEG_CHUNKS, DEG_CHUNKS)], dstb)
    pltpu.sync_copy(w_hbm.at[pl.ds(wid * DEG_CHUNKS, DEG_CHUNKS)], wb)

    def _chunk(j, _):
        pltpu.sync_copy(wb.at[j], deg_s.at[dstb.at[j]], add=True)
        return 0
    lax.fori_loop(0, DEG_CHUNKS, _chunk, 0)
    plsc.subcore_barrier()

    pltpu.sync_copy(deg_s.at[pl.ds(sid * 640, 640)],
                    out_hbm.at[cid, pl.ds(sid * 640, 640)])


def _deg(dst2d, w2d):
    k = pl.kernel(
        _deg_body,
        out_type=jax.ShapeDtypeStruct((NC, NPAD), jnp.float32),
        mesh=_mesh(),
        scratch_types=[
            pltpu.VMEM_SHARED((NPAD,), jnp.float32),
            pltpu.VMEM((DEG_CHUNKS, DEG_K), jnp.int32),
            pltpu.VMEM((DEG_CHUNKS, DEG_K), jnp.float32),
            pltpu.VMEM((640,), jnp.float32),
            pltpu.SemaphoreType.DMA,
        ],
    )
    return k(dst2d, w2d)


def _rsqrt16(d):
    i = plsc.bitcast(d, jnp.int32)
    i = jnp.int32(0x5F3759DF) - lax.shift_right_logical(i, 1)
    y = plsc.bitcast(i, jnp.float32)
    for _ in range(3):
        y = y * (1.5 - 0.5 * d * y * y)
    return y


def _norm_body(deg_hbm, src_hbm, dst_hbm, w_hbm, ne_out, nl_out,
               d0, d1, dinv, srcb, dstb, wb, nb, lb, sem):
    wid = _wid()
    pltpu.sync_copy(deg_hbm.at[0], d0)
    pltpu.sync_copy(deg_hbm.at[1], d1)

    def _dv(i, _):
        sl = pl.ds(i * L, L)
        d = d0[sl] + d1[sl] + 1.0
        dinv[sl] = _rsqrt16(d)
        return 0
    lax.fori_loop(0, NPAD // L, _dv, 0)

    def _lp(i, _):
        sl = pl.ds(i * L, L)
        v = dinv[pl.ds(wid * 320 + i * L, L)]
        lb[sl] = v * v
        return 0
    lax.fori_loop(0, 320 // L, _lp, 0)
    pltpu.sync_copy(lb, nl_out.at[pl.ds(wid * 320, 320)])

    pltpu.sync_copy(src_hbm.at[pl.ds(wid * EPT, EPT)], srcb)
    pltpu.sync_copy(dst_hbm.at[pl.ds(wid * EPT, EPT)], dstb)
    pltpu.sync_copy(w_hbm.at[pl.ds(wid * EPT, EPT)], wb)

    def _ed(i, _):
        sl = pl.ds(i * L, L)
        gs = plsc.load_gather(dinv, [srcb[sl]])
        gd = plsc.load_gather(dinv, [dstb[sl]])
        nb[sl] = gs * wb[sl] * gd
        return 0
    lax.fori_loop(0, EPT // L, _ed, 0)
    pltpu.sync_copy(nb, ne_out.at[pl.ds(wid * EPT, EPT)])


def _norm(deg_parts, src, dst, w):
    k = pl.kernel(
        _norm_body,
        out_type=(jax.ShapeDtypeStruct((E,), jnp.float32),
                  jax.ShapeDtypeStruct((NPAD,), jnp.float32)),
        mesh=_mesh(),
        scratch_types=[
            pltpu.VMEM((NPAD,), jnp.float32),
            pltpu.VMEM((NPAD,), jnp.float32),
            pltpu.VMEM((NPAD,), jnp.float32),
            pltpu.VMEM((EPT,), jnp.int32),
            pltpu.VMEM((EPT,), jnp.int32),
            pltpu.VMEM((EPT,), jnp.float32),
            pltpu.VMEM((EPT,), jnp.float32),
            pltpu.VMEM((320,), jnp.float32),
            pltpu.SemaphoreType.DMA,
        ],
    )
    return k(deg_parts, src, dst, w)


def _agg_body(d, ls_hbm, src_hbm, dst_hbm, nrm_hbm, out_hbm,
              acc_s, srcb, dstb, nrmb, rows, zb, sem):
    cid = lax.axis_index("c")
    sid = lax.axis_index("s")
    wid = _wid()
    nvec = d // L

    def _z(i, _):
        for r in range(nvec):
            zb[i, pl.ds(r * L, L)] = _zero16()
        return 0
    lax.fori_loop(0, 125, _z, 0)
    for q in range(5):
        pltpu.sync_copy(zb, acc_s.at[pl.ds(sid * 625 + q * 125, 125)])
    plsc.subcore_barrier()

    pltpu.sync_copy(src_hbm.at[pl.ds(wid * CHUNKS, CHUNKS)], srcb)
    pltpu.sync_copy(dst_hbm.at[pl.ds(wid * CHUNKS, CHUNKS)], dstb)
    pltpu.sync_copy(nrm_hbm.at[pl.ds(wid * CHUNKS, CHUNKS)], nrmb)

    def _chunk(j, _):
        pltpu.async_copy(ls_hbm.at[srcb.at[j]], rows, sem).wait()

        def _scale(e, _):
            wgt = nrmb[j, e]
            for r in range(nvec):
                sl = pl.ds(r * L, L)
                rows[e, sl] = rows[e, sl] * wgt
            return 0
        lax.fori_loop(0, K, _scale, 0)
        pltpu.sync_copy(rows, acc_s.at[dstb.at[j]], add=True)
        return 0
    lax.fori_loop(0, CHUNKS, _chunk, 0)
    plsc.subcore_barrier()

    pltpu.sync_copy(acc_s.at[pl.ds(sid * 625, 625)],
                    out_hbm.at[cid, pl.ds(sid * 625, 625)])


def _agg(ls, srcF, dstF, nrmF, d):
    k = pl.kernel(
        functools.partial(_agg_body, d),
        out_type=jax.ShapeDtypeStruct((NC, N, d), jnp.float32),
        mesh=_mesh(),
        scratch_types=[
            pltpu.VMEM_SHARED((N, d), jnp.float32),
            pltpu.VMEM((CHUNKS, K), jnp.int32),
            pltpu.VMEM((CHUNKS, K), jnp.int32),
            pltpu.VMEM((CHUNKS, K), jnp.float32),
            pltpu.VMEM((K, d), jnp.float32),
            pltpu.VMEM((125, d), jnp.float32),
            pltpu.SemaphoreType.DMA,
        ],
    )
    return k(ls, srcF, dstF, nrmF)


def _mm_body(x_ref, w_ref, o_ref):
    o_ref[...] = jnp.dot(x_ref[...], w_ref[...],
                         preferred_element_type=jnp.float32)


def _mm(x, w):
    return pl.pallas_call(
        _mm_body,
        out_shape=jax.ShapeDtypeStruct((x.shape[0], w.shape[1]), jnp.float32),
    )(x, w)


def _layer_body(a_ref, b_ref, w_ref, o_ref):
    h = jnp.maximum(a_ref[0] + a_ref[1] + b_ref[...], 0.0)
    o_ref[...] = jnp.dot(h, w_ref[...], preferred_element_type=jnp.float32)


def _layer(acc, b2d, w):
    return pl.pallas_call(
        _layer_body,
        out_shape=jax.ShapeDtypeStruct((N, w.shape[1]), jnp.float32),
    )(acc, b2d, w)


def _final_body(a_ref, b_ref, o_ref):
    o_ref[...] = a_ref[0] + a_ref[1] + b_ref[...]


def _final(acc, b2d):
    return pl.pallas_call(
        _final_body,
        out_shape=jax.ShapeDtypeStruct((N, CP), jnp.float32),
    )(acc, b2d)


def kernel(x, edge_index, edge_attr, W1, b1, W2, b2, W3, b3):
    src = edge_index[0]
    dst = edge_index[1]
    w = edge_attr

    deg_parts = _deg(dst.reshape(E // DEG_K, DEG_K),
                     w.reshape(E // DEG_K, DEG_K))
    norm_e, norm_l = _norm(deg_parts, src, dst, w)

    loop = jnp.arange(N, dtype=jnp.int32)
    ipad = jnp.zeros((EPAD,), jnp.int32)
    srcF = jnp.concatenate([src, loop, ipad]).reshape(NW * CHUNKS, K)
    dstF = jnp.concatenate([dst, loop, ipad]).reshape(NW * CHUNKS, K)
    nrmF = jnp.concatenate([norm_e, norm_l[:N],
                            jnp.zeros((EPAD,), jnp.float32)]
                           ).reshape(NW * CHUNKS, K)

    W3p = jnp.pad(W3, ((0, 0), (0, CP - C)))
    b1r = b1.reshape(1, H)
    b2r = b2.reshape(1, H)
    b3r = jnp.pad(b3, (0, CP - C)).reshape(1, CP)

    lin1 = _mm(x, W1)
    acc1 = _agg(lin1, srcF, dstF, nrmF, H)
    lin2 = _layer(acc1, b1r, W2)
    acc2 = _agg(lin2, srcF, dstF, nrmF, H)
    lin3 = _layer(acc2, b2r, W3p)
    acc3 = _agg(lin3, srcF, dstF, nrmF, CP)
    out = _final(acc3, b3r)
    return out[:, :C]


# speedup vs baseline: 22.6045x; 22.6045x over previous
import functools

import jax
import jax.numpy as jnp
from jax import lax
from jax.experimental import pallas as pl
from jax.experimental.pallas import tpu as pltpu
from jax.experimental.pallas import tpu_sc as plsc

N = 10000
E = 320000
F_IN = 128
H = 128
C = 40
CP = 128

NC = 2
NS = 16
NW = NC * NS
L = 16

NR = 10240
RPT = NR // NS
NA = 10112
RA = NA // NS
K = 128
E2 = E + N
CHUNKS = 4 * (-(-E2 // (NW * K * 4)))
E2P = NW * K * CHUNKS
EPAD = E2P - E2

NSTAGE = 6
STAGE = CHUNKS // NSTAGE
SK = STAGE * K

DEG_K = 125
DEG_CHUNKS = E // (NW * DEG_K)
EPT = E // NW

_mesh = lambda: plsc.VectorSubcoreMesh(core_axis_name="c", subcore_axis_name="s")


def _wid():
    return lax.axis_index("c") * NS + lax.axis_index("s")


def _zero16():
    return jnp.zeros((L,), jnp.float32)


def _deg_body(dst_hbm, w_hbm, out_hbm, deg_s, dstb, wb, zb, sem):
    cid = lax.axis_index("c")
    sid = lax.axis_index("s")
    wid = _wid()

    def _z(i, _):
        zb[pl.ds(i * L, L)] = _zero16()
        return 0
    lax.fori_loop(0, RPT // L, _z, 0)
    pltpu.sync_copy(zb, deg_s.at[pl.ds(sid * RPT, RPT)])
    plsc.subcore_barrier()

    pltpu.sync_copy(dst_hbm.at[wid], dstb)
    pltpu.sync_copy(w_hbm.at[wid], wb)

    def _chunk(j, _):
        pltpu.sync_copy(wb.at[j], deg_s.at[dstb.at[j]], add=True)
        return 0
    lax.fori_loop(0, DEG_CHUNKS, _chunk, 0)
    plsc.subcore_barrier()

    pltpu.sync_copy(deg_s.at[pl.ds(sid * RPT, RPT)],
                    out_hbm.at[pl.ds(cid * NR + sid * RPT, RPT)])


def _deg(dst3, w3):
    k = pl.kernel(
        _deg_body,
        out_type=jax.ShapeDtypeStruct((NC * NR,), jnp.float32),
        mesh=_mesh(),
        compiler_params=pltpu.CompilerParams(needs_layout_passes=False),
        scratch_types=[
            pltpu.VMEM_SHARED((NR,), jnp.float32),
            pltpu.VMEM((DEG_CHUNKS, DEG_K), jnp.int32),
            pltpu.VMEM((DEG_CHUNKS, DEG_K), jnp.float32),
            pltpu.VMEM((RPT,), jnp.float32),
            pltpu.SemaphoreType.DMA,
        ],
    )
    return k(dst3, w3)


def _rsqrt16(d):
    i = lax.bitcast_convert_type(d, jnp.int32)
    i = jnp.int32(0x5F3759DF) - lax.shift_right_logical(i, 1)
    y = lax.bitcast_convert_type(i, jnp.float32)
    for _ in range(3):
        y = y * (1.5 - 0.5 * d * y * y)
    return y


def _norm_body(deg_hbm, src_hbm, dst_hbm, w_hbm, ne_out, nl_out,
               d0, d1, dinv, srcb, dstb, wb, nb, lb, sem):
    wid = _wid()
    pltpu.sync_copy(deg_hbm.at[pl.ds(0, NR)], d0)
    pltpu.sync_copy(deg_hbm.at[pl.ds(NR, NR)], d1)

    def _dv(i, _):
        sl = pl.ds(i * L, L)
        d = d0[sl] + d1[sl] + 1.0
        dinv[sl] = _rsqrt16(d)
        return 0
    lax.fori_loop(0, NR // L, _dv, 0)

    npt = NR // NW

    def _lp(i, _):
        sl = pl.ds(i * L, L)
        v = dinv[pl.ds(wid * npt + i * L, L)]
        lb[sl] = v * v
        return 0
    lax.fori_loop(0, npt // L, _lp, 0)
    pltpu.sync_copy(lb, nl_out.at[pl.ds(wid * npt, npt)])

    pltpu.sync_copy(src_hbm.at[pl.ds(wid * EPT, EPT)], srcb)
    pltpu.sync_copy(dst_hbm.at[pl.ds(wid * EPT, EPT)], dstb)
    pltpu.sync_copy(w_hbm.at[pl.ds(wid * EPT, EPT)], wb)

    def _ed(i, _):
        sl = pl.ds(i * L, L)
        gs = plsc.load_gather(dinv, [srcb[sl]])
        gd = plsc.load_gather(dinv, [dstb[sl]])
        nb[sl] = gs * wb[sl] * gd
        return 0
    lax.fori_loop(0, EPT // L, _ed, 0)
    pltpu.sync_copy(nb, ne_out.at[pl.ds(wid * EPT, EPT)])


def _norm(deg_parts, src, dst, w):
    k = pl.kernel(
        _norm_body,
        out_type=(jax.ShapeDtypeStruct((E,), jnp.float32),
                  jax.ShapeDtypeStruct((NR,), jnp.float32)),
        mesh=_mesh(),
        compiler_params=pltpu.CompilerParams(needs_layout_passes=False),
        scratch_types=[
            pltpu.VMEM((NR,), jnp.float32),
            pltpu.VMEM((NR,), jnp.float32),
            pltpu.VMEM((NR,), jnp.float32),
            pltpu.VMEM((EPT,), jnp.int32),
            pltpu.VMEM((EPT,), jnp.int32),
            pltpu.VMEM((EPT,), jnp.float32),
            pltpu.VMEM((EPT,), jnp.float32),
            pltpu.VMEM((NR // NW,), jnp.float32),
            pltpu.SemaphoreType.DMA,
        ],
    )
    return k(deg_parts, src, dst, w)


def _agg_body(d, ls_hbm, src_hbm, dst_hbm, nrm_hbm, out_hbm,
              acc_s, srcq0, srcq1, dstq0, dstq1, nrmq0, nrmq1, rows0, rows1,
              gsem0, gsem1, ssem0, ssem1, qsem0, qsem1):
    cid = lax.axis_index("c")
    sid = lax.axis_index("s")
    wid = _wid()
    nvec = d // L
    rows = (rows0, rows1)
    gsem = (gsem0, gsem1)
    ssem = (ssem0, ssem1)
    srcq = (srcq0, srcq1)
    dstq = (dstq0, dstq1)
    nrmq = (nrmq0, nrmq1)
    qsem = (qsem0, qsem1)

    def _z(i, _):
        for r in range(nvec):
            rows0[i, pl.ds(r * L, L)] = _zero16()
        return 0
    lax.fori_loop(0, K, _z, 0)
    for q in range(RA // K):
        pltpu.sync_copy(rows0, acc_s.at[pl.ds(sid * RA + q * K, K)])
    pltpu.sync_copy(rows0.at[pl.ds(0, RA % K)],
                    acc_s.at[pl.ds(sid * RA + (RA // K) * K, RA % K)])
    plsc.subcore_barrier()

    def _stagecopies(s, q):
        sl1 = pl.ds(s * SK, SK)
        return (pltpu.make_async_copy(src_hbm.at[wid, 0, sl1], srcq[q],
                                      qsem[q]),
                pltpu.make_async_copy(nrm_hbm.at[wid, 0, sl1], nrmq[q],
                                      qsem[q]),
                pltpu.make_async_copy(dst_hbm.at[wid, s], dstq[q], qsem[q]))

    def _gather(q, jj, b):
        return pltpu.make_async_copy(
            ls_hbm.at[srcq[q].at[pl.ds(jj * K, K)]], rows[b], gsem[b])

    def _scatter(q, jj, b):
        return pltpu.make_async_copy(rows[b], acc_s.at[dstq[q].at[jj]],
                                     ssem[b])

    def _scale(q, b, jj):
        def _g(g, _):
            nv = nrmq[q][pl.ds(jj * K + g * L, L)]
            for i in range(L):
                wgt = nv[i]
                e = g * L + i
                for r in range(nvec):
                    sl = pl.ds(r * L, L)
                    rows[b][e, sl] = rows[b][e, sl] * wgt
            return 0
        lax.fori_loop(0, K // L, _g, 0, unroll=2)

    for c in _stagecopies(0, 0):
        c.start()
    for c in _stagecopies(0, 0):
        c.wait()
    _gather(0, 0, 0).start()

    def _run_stage(t, q):
        s = 2 * t + q

        @pl.when(s >= 1)
        def _():
            _scatter(1 - q, STAGE - 1, 1).wait()
        _gather(q, 1, 1).start()
        _gather(q, 0, 0).wait()
        _scale(q, 0, 0)
        _scatter(q, 0, 0).start(add=True)

        _scatter(q, 0, 0).wait()
        _gather(q, 2, 0).start()
        @pl.when(s + 1 < NSTAGE)
        def _():
            for c in _stagecopies(s + 1, 1 - q):
                c.start()
        _gather(q, 1, 1).wait()
        _scale(q, 1, 1)
        _scatter(q, 1, 1).start(add=True)

        def _mid(p, _):
            j0 = 2 * p
            _scatter(q, j0 - 1, 1).wait()
            _gather(q, j0 + 1, 1).start()
            _gather(q, j0, 0).wait()
            _scale(q, 0, j0)
            _scatter(q, j0, 0).start(add=True)

            _scatter(q, j0, 0).wait()
            _gather(q, j0 + 2, 0).start()
            _gather(q, j0 + 1, 1).wait()
            _scale(q, 1, j0 + 1)
            _scatter(q, j0 + 1, 1).start(add=True)
            return 0
        lax.fori_loop(1, STAGE // 2 - 1, _mid, 0)

        _scatter(q, STAGE - 3, 1).wait()
        _gather(q, STAGE - 1, 1).start()
        _gather(q, STAGE - 2, 0).wait()
        _scale(q, 0, STAGE - 2)
        _scatter(q, STAGE - 2, 0).start(add=True)

        @pl.when(s + 1 < NSTAGE)
        def _():
            _scatter(q, STAGE - 2, 0).wait()
            for c in _stagecopies(s + 1, 1 - q):
                c.wait()
            _gather(1 - q, 0, 0).start()
        _gather(q, STAGE - 1, 1).wait()
        _scale(q, 1, STAGE - 1)
        _scatter(q, STAGE - 1, 1).start(add=True)

    def _super(t, _):
        _run_stage(t, 0)
        _run_stage(t, 1)
        return 0
    lax.fori_loop(0, NSTAGE // 2, _super, 0)

    lastq = (NSTAGE - 1) % 2
    _scatter(lastq, STAGE - 2, 0).wait()
    _scatter(lastq, STAGE - 1, 1).wait()
    plsc.subcore_barrier()

    pltpu.sync_copy(acc_s.at[pl.ds(sid * RA, RA)],
                    out_hbm.at[cid, pl.ds(sid * RA, RA)])


def _agg(ls, srcF, dstF, nrmF, d):
    k = pl.kernel(
        functools.partial(_agg_body, d),
        out_type=jax.ShapeDtypeStruct((NC, NA, d), jnp.float32),
        mesh=_mesh(),
        compiler_params=pltpu.CompilerParams(needs_layout_passes=False),
        scratch_types=[
            pltpu.VMEM_SHARED((NA, d), jnp.float32),
            pltpu.VMEM((SK,), jnp.int32),
            pltpu.VMEM((SK,), jnp.int32),
            pltpu.VMEM((STAGE, K), jnp.int32),
            pltpu.VMEM((STAGE, K), jnp.int32),
            pltpu.VMEM((SK,), jnp.float32),
            pltpu.VMEM((SK,), jnp.float32),
            pltpu.VMEM((K, d), jnp.float32),
            pltpu.VMEM((K, d), jnp.float32),
            pltpu.SemaphoreType.DMA,
            pltpu.SemaphoreType.DMA,
            pltpu.SemaphoreType.DMA,
            pltpu.SemaphoreType.DMA,
            pltpu.SemaphoreType.DMA,
            pltpu.SemaphoreType.DMA,
        ],
    )
    return k(ls, srcF, dstF, nrmF)


def _mm_body(x_ref, w_ref, o_ref):
    o_ref[...] = jnp.dot(x_ref[...], w_ref[...],
                         preferred_element_type=jnp.float32)


def _mm(x, w):
    return pl.pallas_call(
        _mm_body,
        out_shape=jax.ShapeDtypeStruct((x.shape[0], w.shape[1]), jnp.float32),
    )(x, w)


def _layer_body(a_ref, b_ref, w_ref, o_ref):
    h = jnp.maximum(a_ref[0] + a_ref[1] + b_ref[...], 0.0)
    o_ref[...] = jnp.dot(h, w_ref[...], preferred_element_type=jnp.float32)


def _layer(acc, b2d, w):
    return pl.pallas_call(
        _layer_body,
        out_shape=jax.ShapeDtypeStruct((NA, w.shape[1]), jnp.float32),
    )(acc, b2d, w)


def _final_body(a_ref, b_ref, o_ref):
    o_ref[...] = a_ref[0] + a_ref[1] + b_ref[...]


def _final(acc, b2d):
    return pl.pallas_call(
        _final_body,
        out_shape=jax.ShapeDtypeStruct((NA, CP), jnp.float32),
    )(acc, b2d)


def kernel(x, edge_index, edge_attr, W1, b1, W2, b2, W3, b3):
    src = edge_index[0]
    dst = edge_index[1]
    w = edge_attr

    deg_parts = _deg(dst.reshape(NW, DEG_CHUNKS, DEG_K),
                     w.reshape(NW, DEG_CHUNKS, DEG_K))
    norm_e, norm_l = _norm(deg_parts, src, dst, w)

    loop = jnp.arange(N, dtype=jnp.int32)
    ipad = jnp.arange(EPAD, dtype=jnp.int32) % N
    srcF = jnp.concatenate([src, loop, ipad]).reshape(NW, 1, CHUNKS * K)
    dstF = jnp.concatenate([dst, loop, ipad]).reshape(NW, NSTAGE, STAGE, K)
    nrmF = jnp.concatenate([norm_e, norm_l[:N],
                            jnp.zeros((EPAD,), jnp.float32)]
                           ).reshape(NW, 1, CHUNKS * K)

    xp = jnp.pad(x, ((0, NA - N), (0, 0)))
    W3p = jnp.pad(W3, ((0, 0), (0, CP - C)))
    b1r = b1.reshape(1, H)
    b2r = b2.reshape(1, H)
    b3r = jnp.pad(b3, (0, CP - C)).reshape(1, CP)

    lin1 = _mm(xp, W1)
    acc1 = _agg(lin1, srcF, dstF, nrmF, H)
    lin2 = _layer(acc1, b1r, W2)
    acc2 = _agg(lin2, srcF, dstF, nrmF, H)
    lin3 = _layer(acc2, b2r, W3p)
    acc3 = _agg(lin3, srcF, dstF, nrmF, CP)
    out = _final(acc3, b3r)
    return out[:N, :C]


# speedup vs baseline: 22.8480x; 1.0108x over previous
import functools

import jax
import jax.numpy as jnp
from jax import lax
from jax.experimental import pallas as pl
from jax.experimental.pallas import tpu as pltpu
from jax.experimental.pallas import tpu_sc as plsc

N = 10000
E = 320000
F_IN = 128
H = 128
C = 40
CP = 128

NC = 2
NS = 16
NW = NC * NS
L = 16

NR = 10240
RPT = NR // NS
NA = 10112
RA = NA // NS
K = 128
E2 = E + N
CHUNKS = 4 * (-(-E2 // (NW * K * 4)))
E2P = NW * K * CHUNKS
EPAD = E2P - E2

NSTAGE = 6
STAGE = CHUNKS // NSTAGE
SK = STAGE * K

DEG_K = 125
DEG_CHUNKS = E // (NW * DEG_K)
EPT = E // NW

_mesh = lambda: plsc.VectorSubcoreMesh(core_axis_name="c", subcore_axis_name="s")


def _wid():
    return lax.axis_index("c") * NS + lax.axis_index("s")


def _zero16():
    return jnp.zeros((L,), jnp.float32)


def _deg_body(dst_hbm, w_hbm, out_hbm, deg_s, dstb, wb, zb, sem):
    cid = lax.axis_index("c")
    sid = lax.axis_index("s")
    wid = _wid()

    dgs = (pltpu.make_async_copy(dst_hbm.at[wid], dstb, sem),
           pltpu.make_async_copy(w_hbm.at[wid], wb, sem))
    for c in dgs:
        c.start()

    def _z(i, _):
        zb[pl.ds(i * L, L)] = _zero16()
        return 0
    lax.fori_loop(0, RPT // L, _z, 0)
    pltpu.sync_copy(zb, deg_s.at[pl.ds(sid * RPT, RPT)])
    plsc.subcore_barrier()
    for c in dgs:
        c.wait()

    def _chunk(j, _):
        pltpu.sync_copy(wb.at[j], deg_s.at[dstb.at[j]], add=True)
        return 0
    lax.fori_loop(0, DEG_CHUNKS, _chunk, 0)
    plsc.subcore_barrier()

    pltpu.sync_copy(deg_s.at[pl.ds(sid * RPT, RPT)],
                    out_hbm.at[pl.ds(cid * NR + sid * RPT, RPT)])


def _deg(dst3, w3):
    k = pl.kernel(
        _deg_body,
        out_type=jax.ShapeDtypeStruct((NC * NR,), jnp.float32),
        mesh=_mesh(),
        compiler_params=pltpu.CompilerParams(needs_layout_passes=False),
        scratch_types=[
            pltpu.VMEM_SHARED((NR,), jnp.float32),
            pltpu.VMEM((DEG_CHUNKS, DEG_K), jnp.int32),
            pltpu.VMEM((DEG_CHUNKS, DEG_K), jnp.float32),
            pltpu.VMEM((RPT,), jnp.float32),
            pltpu.SemaphoreType.DMA,
        ],
    )
    return k(dst3, w3)


def _rsqrt16(d):
    i = lax.bitcast_convert_type(d, jnp.int32)
    i = jnp.int32(0x5F3759DF) - lax.shift_right_logical(i, 1)
    y = lax.bitcast_convert_type(i, jnp.float32)
    for _ in range(3):
        y = y * (1.5 - 0.5 * d * y * y)
    return y


def _norm_body(deg_hbm, src_hbm, dst_hbm, w_hbm, ne_out, nl_out,
               d0, d1, dinv, srcb, dstb, wb, nb, lb, sem):
    wid = _wid()
    stg = (pltpu.make_async_copy(deg_hbm.at[pl.ds(0, NR)], d0, sem),
           pltpu.make_async_copy(deg_hbm.at[pl.ds(NR, NR)], d1, sem),
           pltpu.make_async_copy(src_hbm.at[pl.ds(wid * EPT, EPT)], srcb,
                                 sem),
           pltpu.make_async_copy(dst_hbm.at[pl.ds(wid * EPT, EPT)], dstb,
                                 sem),
           pltpu.make_async_copy(w_hbm.at[pl.ds(wid * EPT, EPT)], wb, sem))
    for c in stg:
        c.start()
    for c in stg:
        c.wait()

    def _dv(i, _):
        sl = pl.ds(i * L, L)
        d = d0[sl] + d1[sl] + 1.0
        dinv[sl] = _rsqrt16(d)
        return 0
    lax.fori_loop(0, NR // L, _dv, 0)

    npt = NR // NW

    def _lp(i, _):
        sl = pl.ds(i * L, L)
        v = dinv[pl.ds(wid * npt + i * L, L)]
        lb[sl] = v * v
        return 0
    lax.fori_loop(0, npt // L, _lp, 0)
    pltpu.sync_copy(lb, nl_out.at[pl.ds(wid * npt, npt)])

    def _ed(i, _):
        sl = pl.ds(i * L, L)
        gs = plsc.load_gather(dinv, [srcb[sl]])
        gd = plsc.load_gather(dinv, [dstb[sl]])
        nb[sl] = gs * wb[sl] * gd
        return 0
    lax.fori_loop(0, EPT // L, _ed, 0)
    pltpu.sync_copy(nb, ne_out.at[pl.ds(wid * EPT, EPT)])


def _norm(deg_parts, src, dst, w):
    k = pl.kernel(
        _norm_body,
        out_type=(jax.ShapeDtypeStruct((E,), jnp.float32),
                  jax.ShapeDtypeStruct((NR,), jnp.float32)),
        mesh=_mesh(),
        compiler_params=pltpu.CompilerParams(needs_layout_passes=False),
        scratch_types=[
            pltpu.VMEM((NR,), jnp.float32),
            pltpu.VMEM((NR,), jnp.float32),
            pltpu.VMEM((NR,), jnp.float32),
            pltpu.VMEM((EPT,), jnp.int32),
            pltpu.VMEM((EPT,), jnp.int32),
            pltpu.VMEM((EPT,), jnp.float32),
            pltpu.VMEM((EPT,), jnp.float32),
            pltpu.VMEM((NR // NW,), jnp.float32),
            pltpu.SemaphoreType.DMA,
        ],
    )
    return k(deg_parts, src, dst, w)


def _agg_body(d, ls_hbm, src_hbm, dst_hbm, nrm_hbm, out_hbm,
              acc_s, srcq0, srcq1, dstq0, dstq1, nrmq0, nrmq1, rows0, rows1,
              gsem0, gsem1, ssem0, ssem1, qsem0, qsem1):
    cid = lax.axis_index("c")
    sid = lax.axis_index("s")
    wid = _wid()
    nvec = d // L
    rows = (rows0, rows1)
    gsem = (gsem0, gsem1)
    ssem = (ssem0, ssem1)
    srcq = (srcq0, srcq1)
    dstq = (dstq0, dstq1)
    nrmq = (nrmq0, nrmq1)
    qsem = (qsem0, qsem1)

    def _stagecopies(s, q):
        sl1 = pl.ds(s * SK, SK)
        return (pltpu.make_async_copy(src_hbm.at[wid, 0, sl1], srcq[q],
                                      qsem[q]),
                pltpu.make_async_copy(nrm_hbm.at[wid, 0, sl1], nrmq[q],
                                      qsem[q]),
                pltpu.make_async_copy(dst_hbm.at[wid, s], dstq[q], qsem[q]))

    def _gather(q, jj, b):
        return pltpu.make_async_copy(
            ls_hbm.at[srcq[q].at[pl.ds(jj * K, K)]], rows[b], gsem[b])

    def _scatter(q, jj, b):
        return pltpu.make_async_copy(rows[b], acc_s.at[dstq[q].at[jj]],
                                     ssem[b])

    def _scale(q, b, jj):
        def _g(g, _):
            nv = nrmq[q][pl.ds(jj * K + g * L, L)]
            for i in range(L):
                wgt = nv[i]
                e = g * L + i
                for r in range(nvec):
                    sl = pl.ds(r * L, L)
                    rows[b][e, sl] = rows[b][e, sl] * wgt
            return 0
        lax.fori_loop(0, K // L, _g, 0)

    for c in _stagecopies(0, 0):
        c.start()

    def _z(i, _):
        for r in range(nvec):
            rows0[i, pl.ds(r * L, L)] = _zero16()
        return 0
    lax.fori_loop(0, K, _z, 0)
    for q in range(RA // K):
        pltpu.sync_copy(rows0, acc_s.at[pl.ds(sid * RA + q * K, K)])
    pltpu.sync_copy(rows0.at[pl.ds(0, RA % K)],
                    acc_s.at[pl.ds(sid * RA + (RA // K) * K, RA % K)])
    plsc.subcore_barrier()

    for c in _stagecopies(0, 0):
        c.wait()
    _gather(0, 0, 0).start()

    def _run_stage(t, q):
        s = 2 * t + q

        @pl.when(s >= 1)
        def _():
            _scatter(1 - q, STAGE - 1, 1).wait()
        _gather(q, 1, 1).start()
        _gather(q, 0, 0).wait()
        _scale(q, 0, 0)
        _scatter(q, 0, 0).start(add=True)

        _scatter(q, 0, 0).wait()
        _gather(q, 2, 0).start()
        @pl.when(s + 1 < NSTAGE)
        def _():
            for c in _stagecopies(s + 1, 1 - q):
                c.start()
        _gather(q, 1, 1).wait()
        _scale(q, 1, 1)
        _scatter(q, 1, 1).start(add=True)

        def _mid(p, _):
            j0 = 2 * p
            _scatter(q, j0 - 1, 1).wait()
            _gather(q, j0 + 1, 1).start()
            _gather(q, j0, 0).wait()
            _scale(q, 0, j0)
            _scatter(q, j0, 0).start(add=True)

            _scatter(q, j0, 0).wait()
            _gather(q, j0 + 2, 0).start()
            _gather(q, j0 + 1, 1).wait()
            _scale(q, 1, j0 + 1)
            _scatter(q, j0 + 1, 1).start(add=True)
            return 0
        lax.fori_loop(1, STAGE // 2 - 1, _mid, 0)

        _scatter(q, STAGE - 3, 1).wait()
        _gather(q, STAGE - 1, 1).start()
        _gather(q, STAGE - 2, 0).wait()
        _scale(q, 0, STAGE - 2)
        _scatter(q, STAGE - 2, 0).start(add=True)

        @pl.when(s + 1 < NSTAGE)
        def _():
            _scatter(q, STAGE - 2, 0).wait()
            for c in _stagecopies(s + 1, 1 - q):
                c.wait()
            _gather(1 - q, 0, 0).start()
        _gather(q, STAGE - 1, 1).wait()
        _scale(q, 1, STAGE - 1)
        _scatter(q, STAGE - 1, 1).start(add=True)

    def _super(t, _):
        _run_stage(t, 0)
        _run_stage(t, 1)
        return 0
    lax.fori_loop(0, NSTAGE // 2, _super, 0)

    lastq = (NSTAGE - 1) % 2
    _scatter(lastq, STAGE - 2, 0).wait()
    _scatter(lastq, STAGE - 1, 1).wait()
    plsc.subcore_barrier()

    pltpu.sync_copy(acc_s.at[pl.ds(sid * RA, RA)],
                    out_hbm.at[cid, pl.ds(sid * RA, RA)])


def _agg(ls, srcF, dstF, nrmF, d):
    k = pl.kernel(
        functools.partial(_agg_body, d),
        out_type=jax.ShapeDtypeStruct((NC, NA, d), jnp.float32),
        mesh=_mesh(),
        compiler_params=pltpu.CompilerParams(needs_layout_passes=False),
        scratch_types=[
            pltpu.VMEM_SHARED((NA, d), jnp.float32),
            pltpu.VMEM((SK,), jnp.int32),
            pltpu.VMEM((SK,), jnp.int32),
            pltpu.VMEM((STAGE, K), jnp.int32),
            pltpu.VMEM((STAGE, K), jnp.int32),
            pltpu.VMEM((SK,), jnp.float32),
            pltpu.VMEM((SK,), jnp.float32),
            pltpu.VMEM((K, d), jnp.float32),
            pltpu.VMEM((K, d), jnp.float32),
            pltpu.SemaphoreType.DMA,
            pltpu.SemaphoreType.DMA,
            pltpu.SemaphoreType.DMA,
            pltpu.SemaphoreType.DMA,
            pltpu.SemaphoreType.DMA,
            pltpu.SemaphoreType.DMA,
        ],
    )
    return k(ls, srcF, dstF, nrmF)


def _mm_body(x_ref, w_ref, o_ref):
    o_ref[...] = jnp.dot(x_ref[...], w_ref[...],
                         preferred_element_type=jnp.float32)


def _mm(x, w):
    return pl.pallas_call(
        _mm_body,
        out_shape=jax.ShapeDtypeStruct((x.shape[0], w.shape[1]), jnp.float32),
    )(x, w)


def _layer_body(a_ref, b_ref, w_ref, o_ref):
    h = jnp.maximum(a_ref[0] + a_ref[1] + b_ref[...], 0.0)
    o_ref[...] = jnp.dot(h, w_ref[...], preferred_element_type=jnp.float32)


def _layer(acc, b2d, w):
    return pl.pallas_call(
        _layer_body,
        out_shape=jax.ShapeDtypeStruct((NA, w.shape[1]), jnp.float32),
    )(acc, b2d, w)


def _final_body(a_ref, b_ref, o_ref):
    o_ref[...] = a_ref[0] + a_ref[1] + b_ref[...]


def _final(acc, b2d):
    return pl.pallas_call(
        _final_body,
        out_shape=jax.ShapeDtypeStruct((NA, CP), jnp.float32),
    )(acc, b2d)


def kernel(x, edge_index, edge_attr, W1, b1, W2, b2, W3, b3):
    src = edge_index[0]
    dst = edge_index[1]
    w = edge_attr

    deg_parts = _deg(dst.reshape(NW, DEG_CHUNKS, DEG_K),
                     w.reshape(NW, DEG_CHUNKS, DEG_K))
    norm_e, norm_l = _norm(deg_parts, src, dst, w)

    loop = jnp.arange(N, dtype=jnp.int32)
    ipad = jnp.arange(EPAD, dtype=jnp.int32) % N
    srcF = jnp.concatenate([src, loop, ipad]).reshape(NW, 1, CHUNKS * K)
    dstF = jnp.concatenate([dst, loop, ipad]).reshape(NW, NSTAGE, STAGE, K)
    nrmF = jnp.concatenate([norm_e, norm_l[:N],
                            jnp.zeros((EPAD,), jnp.float32)]
                           ).reshape(NW, 1, CHUNKS * K)

    xp = jnp.pad(x, ((0, NA - N), (0, 0)))
    W3p = jnp.pad(W3, ((0, 0), (0, CP - C)))
    b1r = b1.reshape(1, H)
    b2r = b2.reshape(1, H)
    b3r = jnp.pad(b3, (0, CP - C)).reshape(1, CP)

    lin1 = _mm(xp, W1)
    acc1 = _agg(lin1, srcF, dstF, nrmF, H)
    lin2 = _layer(acc1, b1r, W2)
    acc2 = _agg(lin2, srcF, dstF, nrmF, H)
    lin3 = _layer(acc2, b2r, W3p)
    acc3 = _agg(lin3, srcF, dstF, nrmF, CP)
    out = _final(acc3, b3r)
    return out[:N, :C]


# speedup vs baseline: 23.3944x; 1.0239x over previous
import functools

import jax
import jax.numpy as jnp
from jax import lax
from jax.experimental import pallas as pl
from jax.experimental.pallas import tpu as pltpu
from jax.experimental.pallas import tpu_sc as plsc

N = 10000
E = 320000
F_IN = 128
H = 128
C = 40
CP = 128

NC = 2
NS = 16
NW = NC * NS
L = 16

NR = 10240
RPT = NR // NS
NA = 10112
RA = NA // NS
K = 128
E2 = E + N
CHUNKS = 4 * (-(-E2 // (NW * K * 4)))
E2P = NW * K * CHUNKS
EPAD = E2P - E2

NSTAGE = 6
STAGE = CHUNKS // NSTAGE
SK = STAGE * K

DEG_K = 125
DEG_CHUNKS2 = E // (NS * DEG_K)
EPT = E // NW

_mesh = lambda: plsc.VectorSubcoreMesh(core_axis_name="c", subcore_axis_name="s")


def _wid():
    return lax.axis_index("c") * NS + lax.axis_index("s")


def _zero16():
    return jnp.zeros((L,), jnp.float32)


def _rsqrt16(d):
    i = lax.bitcast_convert_type(d, jnp.int32)
    i = jnp.int32(0x5F3759DF) - lax.shift_right_logical(i, 1)
    y = lax.bitcast_convert_type(i, jnp.float32)
    for _ in range(3):
        y = y * (1.5 - 0.5 * d * y * y)
    return y


def _degnorm_body(dw_hbm, ww_hbm, src_hbm, dst_hbm, w_hbm, ne_out, nl_out,
                  deg_s, dgb, wgb, zb, ldeg, dinv, srcb, dstb, wb, nb, lb,
                  sema, semb):
    sid = lax.axis_index("s")
    wid = _wid()

    dgs = (pltpu.make_async_copy(dw_hbm.at[sid], dgb, sema),
           pltpu.make_async_copy(ww_hbm.at[sid], wgb, sema))
    stg = (pltpu.make_async_copy(src_hbm.at[pl.ds(wid * EPT, EPT)], srcb,
                                 semb),
           pltpu.make_async_copy(dst_hbm.at[pl.ds(wid * EPT, EPT)], dstb,
                                 semb),
           pltpu.make_async_copy(w_hbm.at[pl.ds(wid * EPT, EPT)], wb, semb))
    for c in dgs + stg:
        c.start()

    def _z(i, _):
        zb[pl.ds(i * L, L)] = _zero16()
        return 0
    lax.fori_loop(0, RPT // L, _z, 0)
    pltpu.sync_copy(zb, deg_s.at[pl.ds(sid * RPT, RPT)])
    plsc.subcore_barrier()
    for c in dgs:
        c.wait()

    def _chunk(j, _):
        pltpu.sync_copy(wgb.at[j], deg_s.at[dgb.at[j]], add=True)
        return 0
    lax.fori_loop(0, DEG_CHUNKS2, _chunk, 0)
    plsc.subcore_barrier()

    pltpu.sync_copy(deg_s, ldeg)

    def _dv(i, _):
        sl = pl.ds(i * L, L)
        dinv[sl] = _rsqrt16(ldeg[sl] + 1.0)
        return 0
    lax.fori_loop(0, NR // L, _dv, 0)

    npt = NR // NW

    def _lp(i, _):
        sl = pl.ds(i * L, L)
        v = dinv[pl.ds(wid * npt + i * L, L)]
        lb[sl] = v * v
        return 0
    lax.fori_loop(0, npt // L, _lp, 0)
    pltpu.sync_copy(lb, nl_out.at[pl.ds(wid * npt, npt)])

    for c in stg:
        c.wait()

    def _ed(i, _):
        sl = pl.ds(i * L, L)
        gs = plsc.load_gather(dinv, [srcb[sl]])
        gd = plsc.load_gather(dinv, [dstb[sl]])
        nb[sl] = gs * wb[sl] * gd
        return 0
    lax.fori_loop(0, EPT // L, _ed, 0)
    pltpu.sync_copy(nb, ne_out.at[pl.ds(wid * EPT, EPT)])


def _degnorm(dw, ww, src, dst, w):
    k = pl.kernel(
        _degnorm_body,
        out_type=(jax.ShapeDtypeStruct((E,), jnp.float32),
                  jax.ShapeDtypeStruct((NR,), jnp.float32)),
        mesh=_mesh(),
        compiler_params=pltpu.CompilerParams(needs_layout_passes=False),
        scratch_types=[
            pltpu.VMEM_SHARED((NR,), jnp.float32),
            pltpu.VMEM((DEG_CHUNKS2, DEG_K), jnp.int32),
            pltpu.VMEM((DEG_CHUNKS2, DEG_K), jnp.float32),
            pltpu.VMEM((RPT,), jnp.float32),
            pltpu.VMEM((NR,), jnp.float32),
            pltpu.VMEM((NR,), jnp.float32),
            pltpu.VMEM((EPT,), jnp.int32),
            pltpu.VMEM((EPT,), jnp.int32),
            pltpu.VMEM((EPT,), jnp.float32),
            pltpu.VMEM((EPT,), jnp.float32),
            pltpu.VMEM((NR // NW,), jnp.float32),
            pltpu.SemaphoreType.DMA,
            pltpu.SemaphoreType.DMA,
        ],
    )
    return k(dw, ww, src, dst, w)


def _agg_body(d, ls_hbm, src_hbm, dst_hbm, nrm_hbm, out_hbm,
              acc_s, srcq0, srcq1, dstq0, dstq1, nrmq0, nrmq1, rows0, rows1,
              gsem0, gsem1, ssem0, ssem1, qsem0, qsem1):
    cid = lax.axis_index("c")
    sid = lax.axis_index("s")
    wid = _wid()
    nvec = d // L
    rows = (rows0, rows1)
    gsem = (gsem0, gsem1)
    ssem = (ssem0, ssem1)
    srcq = (srcq0, srcq1)
    dstq = (dstq0, dstq1)
    nrmq = (nrmq0, nrmq1)
    qsem = (qsem0, qsem1)

    def _stagecopies(s, q):
        sl1 = pl.ds(s * SK, SK)
        return (pltpu.make_async_copy(src_hbm.at[wid, 0, sl1], srcq[q],
                                      qsem[q]),
                pltpu.make_async_copy(nrm_hbm.at[wid, 0, sl1], nrmq[q],
                                      qsem[q]),
                pltpu.make_async_copy(dst_hbm.at[wid, s], dstq[q], qsem[q]))

    def _gather(q, jj, b):
        return pltpu.make_async_copy(
            ls_hbm.at[srcq[q].at[pl.ds(jj * K, K)]], rows[b], gsem[b])

    def _scatter(q, jj, b):
        return pltpu.make_async_copy(rows[b], acc_s.at[dstq[q].at[jj]],
                                     ssem[b])

    def _scale(q, b, jj):
        def _g(g, _):
            nv = nrmq[q][pl.ds(jj * K + g * L, L)]
            for i in range(L):
                wgt = nv[i]
                e = g * L + i
                for r in range(nvec):
                    sl = pl.ds(r * L, L)
                    rows[b][e, sl] = rows[b][e, sl] * wgt
            return 0
        lax.fori_loop(0, K // L, _g, 0)

    for c in _stagecopies(0, 0):
        c.start()

    def _z(i, _):
        for r in range(nvec):
            rows0[i, pl.ds(r * L, L)] = _zero16()
        return 0
    lax.fori_loop(0, K, _z, 0)
    for q in range(RA // K):
        pltpu.sync_copy(rows0, acc_s.at[pl.ds(sid * RA + q * K, K)])
    pltpu.sync_copy(rows0.at[pl.ds(0, RA % K)],
                    acc_s.at[pl.ds(sid * RA + (RA // K) * K, RA % K)])
    plsc.subcore_barrier()

    for c in _stagecopies(0, 0):
        c.wait()
    _gather(0, 0, 0).start()

    def _run_stage(t, q):
        s = 2 * t + q

        @pl.when(s >= 1)
        def _():
            _scatter(1 - q, STAGE - 1, 1).wait()
        _gather(q, 1, 1).start()
        _gather(q, 0, 0).wait()
        _scale(q, 0, 0)
        _scatter(q, 0, 0).start(add=True)

        _scatter(q, 0, 0).wait()
        _gather(q, 2, 0).start()
        @pl.when(s + 1 < NSTAGE)
        def _():
            for c in _stagecopies(s + 1, 1 - q):
                c.start()
        _gather(q, 1, 1).wait()
        _scale(q, 1, 1)
        _scatter(q, 1, 1).start(add=True)

        def _mid(p, _):
            j0 = 2 * p
            _scatter(q, j0 - 1, 1).wait()
            _gather(q, j0 + 1, 1).start()
            _gather(q, j0, 0).wait()
            _scale(q, 0, j0)
            _scatter(q, j0, 0).start(add=True)

            _scatter(q, j0, 0).wait()
            _gather(q, j0 + 2, 0).start()
            _gather(q, j0 + 1, 1).wait()
            _scale(q, 1, j0 + 1)
            _scatter(q, j0 + 1, 1).start(add=True)
            return 0
        lax.fori_loop(1, STAGE // 2 - 1, _mid, 0)

        _scatter(q, STAGE - 3, 1).wait()
        _gather(q, STAGE - 1, 1).start()
        _gather(q, STAGE - 2, 0).wait()
        _scale(q, 0, STAGE - 2)
        _scatter(q, STAGE - 2, 0).start(add=True)

        @pl.when(s + 1 < NSTAGE)
        def _():
            _scatter(q, STAGE - 2, 0).wait()
            for c in _stagecopies(s + 1, 1 - q):
                c.wait()
            _gather(1 - q, 0, 0).start()
        _gather(q, STAGE - 1, 1).wait()
        _scale(q, 1, STAGE - 1)
        _scatter(q, STAGE - 1, 1).start(add=True)

    def _super(t, _):
        _run_stage(t, 0)
        _run_stage(t, 1)
        return 0
    lax.fori_loop(0, NSTAGE // 2, _super, 0)

    lastq = (NSTAGE - 1) % 2
    _scatter(lastq, STAGE - 2, 0).wait()
    _scatter(lastq, STAGE - 1, 1).wait()
    plsc.subcore_barrier()

    pltpu.sync_copy(acc_s.at[pl.ds(sid * RA, RA)],
                    out_hbm.at[cid, pl.ds(sid * RA, RA)])


def _agg(ls, srcF, dstF, nrmF, d):
    k = pl.kernel(
        functools.partial(_agg_body, d),
        out_type=jax.ShapeDtypeStruct((NC, NA, d), jnp.float32),
        mesh=_mesh(),
        compiler_params=pltpu.CompilerParams(needs_layout_passes=False),
        scratch_types=[
            pltpu.VMEM_SHARED((NA, d), jnp.float32),
            pltpu.VMEM((SK,), jnp.int32),
            pltpu.VMEM((SK,), jnp.int32),
            pltpu.VMEM((STAGE, K), jnp.int32),
            pltpu.VMEM((STAGE, K), jnp.int32),
            pltpu.VMEM((SK,), jnp.float32),
            pltpu.VMEM((SK,), jnp.float32),
            pltpu.VMEM((K, d), jnp.float32),
            pltpu.VMEM((K, d), jnp.float32),
            pltpu.SemaphoreType.DMA,
            pltpu.SemaphoreType.DMA,
            pltpu.SemaphoreType.DMA,
            pltpu.SemaphoreType.DMA,
            pltpu.SemaphoreType.DMA,
            pltpu.SemaphoreType.DMA,
        ],
    )
    return k(ls, srcF, dstF, nrmF)


def _mm_body(x_ref, w_ref, o_ref):
    o_ref[...] = jnp.dot(x_ref[...], w_ref[...],
                         preferred_element_type=jnp.float32)


def _mm(x, w):
    return pl.pallas_call(
        _mm_body,
        out_shape=jax.ShapeDtypeStruct((x.shape[0], w.shape[1]), jnp.float32),
    )(x, w)


def _layer_body(a_ref, b_ref, w_ref, o_ref):
    h = jnp.maximum(a_ref[0] + a_ref[1] + b_ref[...], 0.0)
    o_ref[...] = jnp.dot(h, w_ref[...], preferred_element_type=jnp.float32)


def _layer(acc, b2d, w):
    return pl.pallas_call(
        _layer_body,
        out_shape=jax.ShapeDtypeStruct((NA, w.shape[1]), jnp.float32),
    )(acc, b2d, w)


def _final_body(a_ref, b_ref, o_ref):
    o_ref[...] = a_ref[0] + a_ref[1] + b_ref[...]


def _final(acc, b2d):
    return pl.pallas_call(
        _final_body,
        out_shape=jax.ShapeDtypeStruct((NA, CP), jnp.float32),
    )(acc, b2d)


def kernel(x, edge_index, edge_attr, W1, b1, W2, b2, W3, b3):
    src = edge_index[0]
    dst = edge_index[1]
    w = edge_attr

    norm_e, norm_l = _degnorm(dst.reshape(NS, DEG_CHUNKS2, DEG_K),
                              w.reshape(NS, DEG_CHUNKS2, DEG_K),
                              src, dst, w)

    loop = jnp.arange(N, dtype=jnp.int32)
    ipad = jnp.arange(EPAD, dtype=jnp.int32) % N
    srcF = jnp.concatenate([src, loop, ipad]).reshape(NW, 1, CHUNKS * K)
    dstF = jnp.concatenate([dst, loop, ipad]).reshape(NW, NSTAGE, STAGE, K)
    nrmF = jnp.concatenate([norm_e, norm_l[:N],
                            jnp.zeros((EPAD,), jnp.float32)]
                           ).reshape(NW, 1, CHUNKS * K)

    xp = jnp.pad(x, ((0, NA - N), (0, 0)))
    W3p = jnp.pad(W3, ((0, 0), (0, CP - C)))
    b1r = b1.reshape(1, H)
    b2r = b2.reshape(1, H)
    b3r = jnp.pad(b3, (0, CP - C)).reshape(1, CP)

    lin1 = _mm(xp, W1)
    acc1 = _agg(lin1, srcF, dstF, nrmF, H)
    lin2 = _layer(acc1, b1r, W2)
    acc2 = _agg(lin2, srcF, dstF, nrmF, H)
    lin3 = _layer(acc2, b2r, W3p)
    acc3 = _agg(lin3, srcF, dstF, nrmF, CP)
    out = _final(acc3, b3r)
    return out[:N, :C]


# speedup vs baseline: 23.5235x; 1.0055x over previous
import functools

import jax
import jax.numpy as jnp
from jax import lax
from jax.experimental import pallas as pl
from jax.experimental.pallas import tpu as pltpu
from jax.experimental.pallas import tpu_sc as plsc

N = 10000
E = 320000
F_IN = 128
H = 128
C = 40
CP = 128

NC = 2
NS = 16
NW = NC * NS
L = 16

NR = 10240
RPT = NR // NS
NA = 10112
RA = NA // NS
K = 128
E2 = E + N
CHUNKS = 4 * (-(-E2 // (NW * K * 4)))
E2P = NW * K * CHUNKS
EPAD = E2P - E2

NSTAGE = 6
STAGE = CHUNKS // NSTAGE
SK = STAGE * K

DEG_K = 125
DEG_CHUNKS2 = E // (NS * DEG_K)
EPT = E // NW

_mesh = lambda: plsc.VectorSubcoreMesh(core_axis_name="c", subcore_axis_name="s")


def _wid():
    return lax.axis_index("c") * NS + lax.axis_index("s")


def _zero16():
    return jnp.zeros((L,), jnp.float32)


def _rsqrt16(d):
    i = lax.bitcast_convert_type(d, jnp.int32)
    i = jnp.int32(0x5F3759DF) - lax.shift_right_logical(i, 1)
    y = lax.bitcast_convert_type(i, jnp.float32)
    for _ in range(3):
        y = y * (1.5 - 0.5 * d * y * y)
    return y


def _degnorm_body(dw_hbm, ww_hbm, src_hbm, dst_hbm, w_hbm, ne_out, nl_out,
                  deg_s, dgb, wgb, zb, ldeg, dinv, srcb, dstb, wb, nb, lb,
                  sema, semb):
    sid = lax.axis_index("s")
    wid = _wid()

    dgs = (pltpu.make_async_copy(dw_hbm.at[sid], dgb, sema),
           pltpu.make_async_copy(ww_hbm.at[sid], wgb, sema))
    stg = (pltpu.make_async_copy(src_hbm.at[pl.ds(wid * EPT, EPT)], srcb,
                                 semb),
           pltpu.make_async_copy(dst_hbm.at[pl.ds(wid * EPT, EPT)], dstb,
                                 semb),
           pltpu.make_async_copy(w_hbm.at[pl.ds(wid * EPT, EPT)], wb, semb))
    for c in dgs + stg:
        c.start()

    def _z(i, _):
        zb[pl.ds(i * L, L)] = _zero16()
        return 0
    lax.fori_loop(0, RPT // L, _z, 0)
    pltpu.sync_copy(zb, deg_s.at[pl.ds(sid * RPT, RPT)])
    plsc.subcore_barrier()
    for c in dgs:
        c.wait()

    def _chunk(g, _):
        batch = [pltpu.make_async_copy(wgb.at[g * 8 + jj],
                                       deg_s.at[dgb.at[g * 8 + jj]], sema)
                 for jj in range(8)]
        for c in batch:
            c.start(add=True)
        for c in batch:
            c.wait()
        return 0
    lax.fori_loop(0, DEG_CHUNKS2 // 8, _chunk, 0)
    plsc.subcore_barrier()

    pltpu.sync_copy(deg_s, ldeg)

    def _dv(i, _):
        sl = pl.ds(i * L, L)
        dinv[sl] = _rsqrt16(ldeg[sl] + 1.0)
        return 0
    lax.fori_loop(0, NR // L, _dv, 0)

    npt = NR // NW

    def _lp(i, _):
        sl = pl.ds(i * L, L)
        v = dinv[pl.ds(wid * npt + i * L, L)]
        lb[sl] = v * v
        return 0
    lax.fori_loop(0, npt // L, _lp, 0)
    pltpu.sync_copy(lb, nl_out.at[pl.ds(wid * npt, npt)])

    for c in stg:
        c.wait()

    def _ed(i, _):
        sl = pl.ds(i * L, L)
        gs = plsc.load_gather(dinv, [srcb[sl]])
        gd = plsc.load_gather(dinv, [dstb[sl]])
        nb[sl] = gs * wb[sl] * gd
        return 0
    lax.fori_loop(0, EPT // L, _ed, 0)
    pltpu.sync_copy(nb, ne_out.at[pl.ds(wid * EPT, EPT)])


def _degnorm(dw, ww, src, dst, w):
    k = pl.kernel(
        _degnorm_body,
        out_type=(jax.ShapeDtypeStruct((E,), jnp.float32),
                  jax.ShapeDtypeStruct((NR,), jnp.float32)),
        mesh=_mesh(),
        compiler_params=pltpu.CompilerParams(needs_layout_passes=False),
        scratch_types=[
            pltpu.VMEM_SHARED((NR,), jnp.float32),
            pltpu.VMEM((DEG_CHUNKS2, DEG_K), jnp.int32),
            pltpu.VMEM((DEG_CHUNKS2, DEG_K), jnp.float32),
            pltpu.VMEM((RPT,), jnp.float32),
            pltpu.VMEM((NR,), jnp.float32),
            pltpu.VMEM((NR,), jnp.float32),
            pltpu.VMEM((EPT,), jnp.int32),
            pltpu.VMEM((EPT,), jnp.int32),
            pltpu.VMEM((EPT,), jnp.float32),
            pltpu.VMEM((EPT,), jnp.float32),
            pltpu.VMEM((NR // NW,), jnp.float32),
            pltpu.SemaphoreType.DMA,
            pltpu.SemaphoreType.DMA,
        ],
    )
    return k(dw, ww, src, dst, w)


def _agg_body(d, ls_hbm, src_hbm, dst_hbm, nrm_hbm, out_hbm,
              acc_s, srcq0, srcq1, dstq0, dstq1, nrmq0, nrmq1, rows0, rows1,
              gsem0, gsem1, ssem0, ssem1, qsem0, qsem1):
    cid = lax.axis_index("c")
    sid = lax.axis_index("s")
    wid = _wid()
    nvec = d // L
    rows = (rows0, rows1)
    gsem = (gsem0, gsem1)
    ssem = (ssem0, ssem1)
    srcq = (srcq0, srcq1)
    dstq = (dstq0, dstq1)
    nrmq = (nrmq0, nrmq1)
    qsem = (qsem0, qsem1)

    def _stagecopies(s, q):
        sl1 = pl.ds(s * SK, SK)
        return (pltpu.make_async_copy(src_hbm.at[wid, 0, sl1], srcq[q],
                                      qsem[q]),
                pltpu.make_async_copy(nrm_hbm.at[wid, 0, sl1], nrmq[q],
                                      qsem[q]),
                pltpu.make_async_copy(dst_hbm.at[wid, s], dstq[q], qsem[q]))

    def _gather(q, jj, b):
        return pltpu.make_async_copy(
            ls_hbm.at[srcq[q].at[pl.ds(jj * K, K)]], rows[b], gsem[b])

    def _scatter(q, jj, b):
        return pltpu.make_async_copy(rows[b], acc_s.at[dstq[q].at[jj]],
                                     ssem[b])

    def _scale(q, b, jj):
        def _g(g, _):
            nv = nrmq[q][pl.ds(jj * K + g * L, L)]
            for i in range(L):
                wgt = nv[i]
                e = g * L + i
                for r in range(nvec):
                    sl = pl.ds(r * L, L)
                    rows[b][e, sl] = rows[b][e, sl] * wgt
            return 0
        lax.fori_loop(0, K // L, _g, 0)

    for c in _stagecopies(0, 0):
        c.start()

    def _z(i, _):
        for r in range(nvec):
            rows0[i, pl.ds(r * L, L)] = _zero16()
        return 0
    lax.fori_loop(0, K, _z, 0)
    for q in range(RA // K):
        pltpu.sync_copy(rows0, acc_s.at[pl.ds(sid * RA + q * K, K)])
    pltpu.sync_copy(rows0.at[pl.ds(0, RA % K)],
                    acc_s.at[pl.ds(sid * RA + (RA // K) * K, RA % K)])
    plsc.subcore_barrier()

    for c in _stagecopies(0, 0):
        c.wait()
    _gather(0, 0, 0).start()

    def _run_stage(t, q):
        s = 2 * t + q

        @pl.when(s >= 1)
        def _():
            _scatter(1 - q, STAGE - 1, 1).wait()
        _gather(q, 1, 1).start()
        _gather(q, 0, 0).wait()
        _scale(q, 0, 0)
        _scatter(q, 0, 0).start(add=True)

        _scatter(q, 0, 0).wait()
        _gather(q, 2, 0).start()
        @pl.when(s + 1 < NSTAGE)
        def _():
            for c in _stagecopies(s + 1, 1 - q):
                c.start()
        _gather(q, 1, 1).wait()
        _scale(q, 1, 1)
        _scatter(q, 1, 1).start(add=True)

        def _mid(p, _):
            j0 = 2 * p
            _scatter(q, j0 - 1, 1).wait()
            _gather(q, j0 + 1, 1).start()
            _gather(q, j0, 0).wait()
            _scale(q, 0, j0)
            _scatter(q, j0, 0).start(add=True)

            _scatter(q, j0, 0).wait()
            _gather(q, j0 + 2, 0).start()
            _gather(q, j0 + 1, 1).wait()
            _scale(q, 1, j0 + 1)
            _scatter(q, j0 + 1, 1).start(add=True)
            return 0
        lax.fori_loop(1, STAGE // 2 - 1, _mid, 0)

        _scatter(q, STAGE - 3, 1).wait()
        _gather(q, STAGE - 1, 1).start()
        _gather(q, STAGE - 2, 0).wait()
        _scale(q, 0, STAGE - 2)
        _scatter(q, STAGE - 2, 0).start(add=True)

        @pl.when(s + 1 < NSTAGE)
        def _():
            _scatter(q, STAGE - 2, 0).wait()
            for c in _stagecopies(s + 1, 1 - q):
                c.wait()
            _gather(1 - q, 0, 0).start()
        _gather(q, STAGE - 1, 1).wait()
        _scale(q, 1, STAGE - 1)
        _scatter(q, STAGE - 1, 1).start(add=True)

    def _super(t, _):
        _run_stage(t, 0)
        _run_stage(t, 1)
        return 0
    lax.fori_loop(0, NSTAGE // 2, _super, 0)

    lastq = (NSTAGE - 1) % 2
    _scatter(lastq, STAGE - 2, 0).wait()
    _scatter(lastq, STAGE - 1, 1).wait()
    plsc.subcore_barrier()

    pltpu.sync_copy(acc_s.at[pl.ds(sid * RA, RA)],
                    out_hbm.at[cid, pl.ds(sid * RA, RA)])


def _agg(ls, srcF, dstF, nrmF, d):
    k = pl.kernel(
        functools.partial(_agg_body, d),
        out_type=jax.ShapeDtypeStruct((NC, NA, d), jnp.float32),
        mesh=_mesh(),
        compiler_params=pltpu.CompilerParams(needs_layout_passes=False),
        scratch_types=[
            pltpu.VMEM_SHARED((NA, d), jnp.float32),
            pltpu.VMEM((SK,), jnp.int32),
            pltpu.VMEM((SK,), jnp.int32),
            pltpu.VMEM((STAGE, K), jnp.int32),
            pltpu.VMEM((STAGE, K), jnp.int32),
            pltpu.VMEM((SK,), jnp.float32),
            pltpu.VMEM((SK,), jnp.float32),
            pltpu.VMEM((K, d), jnp.float32),
            pltpu.VMEM((K, d), jnp.float32),
            pltpu.SemaphoreType.DMA,
            pltpu.SemaphoreType.DMA,
            pltpu.SemaphoreType.DMA,
            pltpu.SemaphoreType.DMA,
            pltpu.SemaphoreType.DMA,
            pltpu.SemaphoreType.DMA,
        ],
    )
    return k(ls, srcF, dstF, nrmF)


def _mm_body(x_ref, w_ref, o_ref):
    o_ref[...] = jnp.dot(x_ref[...], w_ref[...],
                         preferred_element_type=jnp.float32)


def _mm(x, w):
    return pl.pallas_call(
        _mm_body,
        out_shape=jax.ShapeDtypeStruct((x.shape[0], w.shape[1]), jnp.float32),
    )(x, w)


def _layer_body(a_ref, b_ref, w_ref, o_ref):
    h = jnp.maximum(a_ref[0] + a_ref[1] + b_ref[...], 0.0)
    o_ref[...] = jnp.dot(h, w_ref[...], preferred_element_type=jnp.float32)


def _layer(acc, b2d, w):
    return pl.pallas_call(
        _layer_body,
        out_shape=jax.ShapeDtypeStruct((NA, w.shape[1]), jnp.float32),
    )(acc, b2d, w)


def _final_body(a_ref, b_ref, o_ref):
    o_ref[...] = a_ref[0] + a_ref[1] + b_ref[...]


def _final(acc, b2d):
    return pl.pallas_call(
        _final_body,
        out_shape=jax.ShapeDtypeStruct((NA, CP), jnp.float32),
    )(acc, b2d)


def kernel(x, edge_index, edge_attr, W1, b1, W2, b2, W3, b3):
    src = edge_index[0]
    dst = edge_index[1]
    w = edge_attr

    norm_e, norm_l = _degnorm(dst.reshape(NS, DEG_CHUNKS2, DEG_K),
                              w.reshape(NS, DEG_CHUNKS2, DEG_K),
                              src, dst, w)

    loop = jnp.arange(N, dtype=jnp.int32)
    ipad = jnp.arange(EPAD, dtype=jnp.int32) % N
    srcF = jnp.concatenate([src, loop, ipad]).reshape(NW, 1, CHUNKS * K)
    dstF = jnp.concatenate([dst, loop, ipad]).reshape(NW, NSTAGE, STAGE, K)
    nrmF = jnp.concatenate([norm_e, norm_l[:N],
                            jnp.zeros((EPAD,), jnp.float32)]
                           ).reshape(NW, 1, CHUNKS * K)

    xp = jnp.pad(x, ((0, NA - N), (0, 0)))
    W3p = jnp.pad(W3, ((0, 0), (0, CP - C)))
    b1r = b1.reshape(1, H)
    b2r = b2.reshape(1, H)
    b3r = jnp.pad(b3, (0, CP - C)).reshape(1, CP)

    lin1 = _mm(xp, W1)
    acc1 = _agg(lin1, srcF, dstF, nrmF, H)
    lin2 = _layer(acc1, b1r, W2)
    acc2 = _agg(lin2, srcF, dstF, nrmF, H)
    lin3 = _layer(acc2, b2r, W3p)
    acc3 = _agg(lin3, srcF, dstF, nrmF, CP)
    out = _final(acc3, b3r)
    return out[:N, :C]
